# baseline (speedup 1.0000x reference)
import dataclasses

import jax
import jax.numpy as jnp
from jax import lax
from jax.experimental import pallas as pl
from jax.experimental.pallas import tpu as pltpu
from jax.experimental.pallas import tpu_sc as plsc

_L = 16
_NT = 32

_F_CH = 896
_F_CHUNKS = 7
_F_IDX_ROWS = _F_CH // 128
_F_PER_TILE = _F_CH * _F_CHUNKS
_F_PAD = _F_PER_TILE * _NT

_V_CH = 448
_V_CHUNKS = 7
_V_IDX_ROWS = _V_CH * 8 // 128
_V_PER_TILE = _V_CH * _V_CHUNKS
_N_PAD = _V_PER_TILE * _NT


def _c16(v):
    return jnp.full((_L,), v, jnp.int32)


def _rsqrt(s):
    i = plsc.bitcast(s, jnp.int32)
    y = plsc.bitcast(jnp.int32(0x5F3759DF) - (i >> 1), jnp.float32)
    xh = s * 0.5
    for _ in range(3):
        y = y * (1.5 - xh * y * y)
    return y


def _wid():
    return lax.axis_index("s") * 2 + lax.axis_index("c")


def _rows_body(pp_hbm, rows_hbm, src, outb, sem):
    wid = _wid()
    iota = lax.iota(jnp.int32, _L)
    vt3 = _V_PER_TILE * 3
    cps = [
        pltpu.async_copy(
            pp_hbm.at[pl.ds(b * (_N_PAD * 3) + wid * vt3, vt3)],
            src.at[pl.ds(b * vt3, vt3)],
            sem,
        )
        for b in range(4)
    ]
    for c in cps:
        c.wait()

    @pl.loop(0, _V_PER_TILE // _L)
    def _grp(g):
        riv = g * _L + iota
        r3 = riv * 3
        for b in range(4):
            for k in range(3):
                val = plsc.load_gather(src, [b * vt3 + r3 + k])
                plsc.store_scatter(outb, [riv, _c16(3 * b + k)], val)

    pltpu.sync_copy(outb, rows_hbm.at[pl.ds(wid * _V_PER_TILE, _V_PER_TILE)])


def _face_body(p_hbm, f0, f1, f2, fn_hbm, idx0, idx1, idx2, r0, r1, r2, outb,
               sem):
    wid = _wid()
    iota = lax.iota(jnp.int32, _L)

    @pl.loop(0, _F_CHUNKS)
    def _chunk(ch):
        cidx = wid * _F_CHUNKS + ch
        base = cidx * _F_CH
        cps = [
            pltpu.async_copy(f0.at[pl.ds(base, _F_CH)], idx0, sem),
            pltpu.async_copy(f1.at[pl.ds(base, _F_CH)], idx1, sem),
            pltpu.async_copy(f2.at[pl.ds(base, _F_CH)], idx2, sem),
        ]
        for c in cps:
            c.wait()
        cps = [
            pltpu.async_copy(p_hbm.at[idx0], r0, sem),
            pltpu.async_copy(p_hbm.at[idx1], r1, sem),
            pltpu.async_copy(p_hbm.at[idx2], r2, sem),
        ]
        for c in cps:
            c.wait()

        @pl.loop(0, _F_CH // _L)
        def _grp(g):
            riv = g * _L + iota
            a0 = [plsc.load_gather(r0, [riv, _c16(j)]) for j in range(12)]
            a1 = [plsc.load_gather(r1, [riv, _c16(j)]) for j in range(12)]
            a2 = [plsc.load_gather(r2, [riv, _c16(j)]) for j in range(12)]
            u = [a1[j] - a0[j] for j in range(12)]
            v = [a2[j] - a0[j] for j in range(12)]
            for b in range(4):
                X, Y, Z = 3 * b, 3 * b + 1, 3 * b + 2
                nx = u[Y] * v[Z] - u[Z] * v[Y]
                ny = u[Z] * v[X] - u[X] * v[Z]
                nz = u[X] * v[Y] - u[Y] * v[X]
                s = jnp.maximum(nx * nx + ny * ny + nz * nz, 1e-24)
                r = _rsqrt(s)
                plsc.store_scatter(outb, [riv, _c16(X)], nx * r)
                plsc.store_scatter(outb, [riv, _c16(Y)], ny * r)
                plsc.store_scatter(outb, [riv, _c16(Z)], nz * r)

        pltpu.sync_copy(outb, fn_hbm.at[pl.ds(base, _F_CH)])


def _vert_body(fn_hbm, vt, w_hbm, vn_hbm, idxb, wb, rows, ob0, ob1, ob2, ob3,
               sem):
    wid = _wid()
    iota = lax.iota(jnp.int32, _L)
    obs = [ob0, ob1, ob2, ob3]

    @pl.loop(0, _V_CHUNKS)
    def _chunk(ch):
        cidx = wid * _V_CHUNKS + ch
        cps = [
            pltpu.async_copy(
                vt.at[pl.ds(cidx * _V_CH * 8, _V_CH * 8)], idxb, sem
            ),
            pltpu.async_copy(
                w_hbm.at[pl.ds(cidx * _V_CH * 8, _V_CH * 8)], wb, sem
            ),
        ]
        for c in cps:
            c.wait()
        pltpu.async_copy(fn_hbm.at[idxb], rows, sem).wait()

        @pl.loop(0, _V_CH // _L)
        def _grp(g):
            riv = g * _L + iota
            r3 = riv * 3
            rb = riv * 8
            rix = [rb + c for c in range(8)]
            ws = [plsc.load_gather(wb, [rix[c]]) for c in range(8)]
            for b in range(4):
                comp = []
                for k in range(3):
                    cj = _c16(3 * b + k)
                    t = ws[0] * plsc.load_gather(rows, [rix[0], cj])
                    for c in range(1, 8):
                        t = t + ws[c] * plsc.load_gather(rows, [rix[c], cj])
                    comp.append(t)
                s = jnp.maximum(
                    comp[0] * comp[0] + comp[1] * comp[1] + comp[2] * comp[2],
                    1e-24,
                )
                r = _rsqrt(s)
                for k in range(3):
                    plsc.store_scatter(obs[b], [r3 + k], comp[k] * r)

        for b in range(4):
            pltpu.sync_copy(
                obs[b],
                vn_hbm.at[pl.ds(b * (_N_PAD * 3) + cidx * _V_CH * 3,
                                _V_CH * 3)],
            )


def _center_body(x_ref, o_ref, n):
    x = x_ref[...]
    rows, cols = x.shape
    flat = (
        lax.broadcasted_iota(jnp.int32, (rows, cols), 0) * cols
        + lax.broadcasted_iota(jnp.int32, (rows, cols), 1)
    )
    bidx = flat // (n * 3)
    cidx = flat % 3
    inv_n = 1.0 / n
    zero = jnp.zeros_like(x)
    sub = None
    for b in range(4):
        ms = []
        for k in range(3):
            msk = (bidx == b) & (cidx == k)
            ms.append(jnp.sum(jnp.where(msk, x, zero)) * inv_n)
        mb = jnp.where(cidx == 0, ms[0], jnp.where(cidx == 1, ms[1], ms[2]))
        sub = mb if sub is None else jnp.where(bidx == b, mb, sub)
    o_ref[...] = x - sub


def kernel(points, faces, vert_tri_indices, vert_tri_weights):
    bs, n, _ = points.shape
    f = faces.shape[0]
    dt = points.dtype

    pp_flat = jnp.pad(
        points.reshape(bs, n * 3), ((0, 0), (0, (_N_PAD - n) * 3))
    ).reshape(-1)
    f1d = [jnp.pad(faces[:, k], (0, _F_PAD - f)) for k in range(3)]
    vt1d = jnp.pad(vert_tri_indices.reshape(-1), (0, (_N_PAD - n) * 8))
    w_flat = jnp.pad(vert_tri_weights.reshape(-1), (0, (_N_PAD - n) * 8))

    mesh = plsc.VectorSubcoreMesh(core_axis_name="c", subcore_axis_name="s")
    f32 = jnp.float32
    i32 = jnp.int32
    cp = pltpu.CompilerParams()
    fields = pltpu.CompilerParams.__dataclass_fields__
    if "needs_layout_passes" in fields:
        cp = dataclasses.replace(cp, needs_layout_passes=False)
    if "use_tc_tiling_on_sc" in fields:
        cp = dataclasses.replace(cp, use_tc_tiling_on_sc=False)

    p_rows = pl.kernel(
        _rows_body,
        out_type=jax.ShapeDtypeStruct((_N_PAD, 16), f32),
        mesh=mesh,
        scratch_types=[
            pltpu.VMEM((_V_PER_TILE * 3 * 4,), f32),
            pltpu.VMEM((_V_PER_TILE, 16), f32),
            pltpu.SemaphoreType.DMA,
        ],
        compiler_params=cp,
    )(pp_flat)

    fn_rows = pl.kernel(
        _face_body,
        out_type=jax.ShapeDtypeStruct((_F_PAD, 16), f32),
        mesh=mesh,
        scratch_types=[
            pltpu.VMEM((_F_CH,), i32),
            pltpu.VMEM((_F_CH,), i32),
            pltpu.VMEM((_F_CH,), i32),
            pltpu.VMEM((_F_CH, 16), f32),
            pltpu.VMEM((_F_CH, 16), f32),
            pltpu.VMEM((_F_CH, 16), f32),
            pltpu.VMEM((_F_CH, 16), f32),
            pltpu.SemaphoreType.DMA,
        ],
        compiler_params=cp,
    )(p_rows, f1d[0], f1d[1], f1d[2])

    vn_flat = pl.kernel(
        _vert_body,
        out_type=jax.ShapeDtypeStruct((4 * _N_PAD * 3,), f32),
        mesh=mesh,
        scratch_types=[
            pltpu.VMEM((_V_CH * 8,), i32),
            pltpu.VMEM((_V_CH * 8,), f32),
            pltpu.VMEM((_V_CH * 8, 16), f32),
            pltpu.VMEM((_V_CH * 3,), f32),
            pltpu.VMEM((_V_CH * 3,), f32),
            pltpu.VMEM((_V_CH * 3,), f32),
            pltpu.VMEM((_V_CH * 3,), f32),
            pltpu.SemaphoreType.DMA,
        ],
        compiler_params=cp,
    )(fn_rows, vt1d, w_flat)

    pts2d = pl.pallas_call(
        lambda x_ref, o_ref: _center_body(x_ref, o_ref, n),
        out_shape=jax.ShapeDtypeStruct((bs * n * 3 // 128, 128), f32),
    )(points.reshape(bs * n * 3 // 128, 128))

    pts = pts2d.reshape(bs, n, 3)
    vn = vn_flat.reshape(bs, _N_PAD * 3)[:, : n * 3].reshape(bs, n, 3)
    rgb = jnp.ones((bs, n, 3), dt)
    return pts, vn, rgb


# speedup vs baseline: 2.5466x; 2.5466x over previous
import dataclasses

import jax
import jax.numpy as jnp
from jax import lax
from jax.experimental import pallas as pl
from jax.experimental.pallas import tpu as pltpu
from jax.experimental.pallas import tpu_sc as plsc

_L = 16
_NT = 32

_F_CH = 896
_F_CHUNKS = 7
_F_PER_TILE = _F_CH * _F_CHUNKS
_F_PAD = _F_PER_TILE * _NT

_V_CH = 224
_V_CHUNKS = 14
_V_PER_TILE = _V_CH * _V_CHUNKS
_N_PAD = _V_PER_TILE * _NT


def _c16(v):
    return jnp.full((_L,), v, jnp.int32)


def _rsqrt(s):
    i = plsc.bitcast(s, jnp.int32)
    y = plsc.bitcast(jnp.int32(0x5F3759DF) - (i >> 1), jnp.float32)
    xh = s * 0.5
    for _ in range(3):
        y = y * (1.5 - xh * y * y)
    return y


def _wid():
    return lax.axis_index("s") * 2 + lax.axis_index("c")


def _face_body(p_hbm, f0, f1, f2, fn_hbm, idx0, idx1, idx2, r0, r1, r2, outb,
               sem):
    wid = _wid()
    iota = lax.iota(jnp.int32, _L)

    @pl.loop(0, _F_CHUNKS)
    def _chunk(ch):
        cidx = wid * _F_CHUNKS + ch
        base = cidx * _F_CH
        cps = [
            pltpu.async_copy(f0.at[pl.ds(base, _F_CH)], idx0, sem),
            pltpu.async_copy(f1.at[pl.ds(base, _F_CH)], idx1, sem),
            pltpu.async_copy(f2.at[pl.ds(base, _F_CH)], idx2, sem),
        ]
        for c in cps:
            c.wait()
        cps = [
            pltpu.async_copy(p_hbm.at[idx0], r0, sem),
            pltpu.async_copy(p_hbm.at[idx1], r1, sem),
            pltpu.async_copy(p_hbm.at[idx2], r2, sem),
        ]
        for c in cps:
            c.wait()

        @pl.loop(0, _F_CH // _L)
        def _grp(g):
            riv = g * _L + iota
            a0 = [plsc.load_gather(r0, [riv, _c16(j)]) for j in range(12)]
            a1 = [plsc.load_gather(r1, [riv, _c16(j)]) for j in range(12)]
            a2 = [plsc.load_gather(r2, [riv, _c16(j)]) for j in range(12)]
            u = [a1[j] - a0[j] for j in range(12)]
            v = [a2[j] - a0[j] for j in range(12)]
            for b in range(4):
                X, Y, Z = 3 * b, 3 * b + 1, 3 * b + 2
                nx = u[Y] * v[Z] - u[Z] * v[Y]
                ny = u[Z] * v[X] - u[X] * v[Z]
                nz = u[X] * v[Y] - u[Y] * v[X]
                s = jnp.maximum(nx * nx + ny * ny + nz * nz, 1e-24)
                r = _rsqrt(s)
                plsc.store_scatter(outb, [riv, _c16(X)], nx * r)
                plsc.store_scatter(outb, [riv, _c16(Y)], ny * r)
                plsc.store_scatter(outb, [riv, _c16(Z)], nz * r)

        pltpu.sync_copy(outb, fn_hbm.at[pl.ds(base, _F_CH)])


def _vert_body(fn_hbm, vt, w_hbm, vn_hbm,
               idx0, idx1, wb0, wb1, rows0, rows1, outb0, outb1,
               semi0, semi1, semg0, semg1):
    wid = _wid()
    iota = lax.iota(jnp.int32, _L)
    c8 = _V_CH * 8
    last = _V_CHUNKS - 1

    def fire_idxw(ch, idxr, wr, sem):
        cidx = wid * _V_CHUNKS + ch
        off = pl.multiple_of(cidx * c8, 8)
        pltpu.async_copy(vt.at[pl.ds(off, c8)], idxr, sem)
        pltpu.async_copy(w_hbm.at[pl.ds(off, c8)], wr, sem)

    def wait_idxw(idxr, wr, sem):
        pltpu.make_async_copy(vt.at[pl.ds(0, c8)], idxr, sem).wait()
        pltpu.make_async_copy(w_hbm.at[pl.ds(0, c8)], wr, sem).wait()

    def fire_g(idxr, rowsr, sem):
        pltpu.async_copy(fn_hbm.at[idxr], rowsr, sem)

    def wait_g(idxr, rowsr, sem):
        pltpu.make_async_copy(fn_hbm.at[idxr], rowsr, sem).wait()

    def compute(ch, wr, rowsr, outr):
        @pl.loop(0, _V_CH // _L)
        def _grp(g):
            riv = g * _L + iota
            rb = riv * 8
            rix = [rb + c for c in range(8)]
            ws = [plsc.load_gather(wr, [rix[c]]) for c in range(8)]
            for b in range(4):
                comp = []
                for k in range(3):
                    cj = _c16(3 * b + k)
                    t = ws[0] * plsc.load_gather(rowsr, [rix[0], cj])
                    for c in range(1, 8):
                        t = t + ws[c] * plsc.load_gather(rowsr, [rix[c], cj])
                    comp.append(t)
                s = jnp.maximum(
                    comp[0] * comp[0] + comp[1] * comp[1] + comp[2] * comp[2],
                    1e-24,
                )
                r = _rsqrt(s)
                for k in range(3):
                    plsc.store_scatter(outr, [riv, _c16(3 * b + k)],
                                       comp[k] * r)

        cidx = wid * _V_CHUNKS + ch
        off = pl.multiple_of(cidx * _V_CH, 8)
        pltpu.sync_copy(outr, vn_hbm.at[pl.ds(off, _V_CH)])

    fire_idxw(0, idx0, wb0, semi0)
    wait_idxw(idx0, wb0, semi0)
    fire_g(idx0, rows0, semg0)
    fire_idxw(1, idx1, wb1, semi1)

    @pl.loop(0, _V_CHUNKS // 2)
    def _pair(i):
        e = i * 2
        wait_g(idx0, rows0, semg0)
        wait_idxw(idx1, wb1, semi1)
        fire_g(idx1, rows1, semg1)
        compute(e, wb0, rows0, outb0)
        fire_idxw(jnp.minimum(e + 2, last), idx0, wb0, semi0)
        wait_g(idx1, rows1, semg1)
        wait_idxw(idx0, wb0, semi0)
        fire_g(idx0, rows0, semg0)
        compute(e + 1, wb1, rows1, outb1)
        fire_idxw(jnp.minimum(e + 3, last), idx1, wb1, semi1)

    wait_g(idx0, rows0, semg0)
    wait_idxw(idx1, wb1, semi1)


def _center_body(x_ref, o_ref, n):
    x = x_ref[...]
    s = jnp.sum(x, axis=0, keepdims=True)
    m = s[:, 0:16]
    for k in range(1, 8):
        m = m + s[:, 16 * k:16 * (k + 1)]
    m = m * (1.0 / n)
    o_ref[...] = x - jnp.concatenate([m] * 8, axis=1)


def kernel(points, faces, vert_tri_indices, vert_tri_weights):
    bs, n, _ = points.shape
    f = faces.shape[0]
    dt = points.dtype

    p_rows = jnp.transpose(points, (1, 0, 2)).reshape(n, bs * 3)
    p_rows = jnp.pad(p_rows, ((0, 0), (0, 16 - bs * 3)))
    f1d = [jnp.pad(faces[:, k], (0, _F_PAD - f)) for k in range(3)]
    vt1d = jnp.pad(vert_tri_indices.reshape(-1), (0, (_N_PAD - n) * 8))
    w_flat = jnp.pad(vert_tri_weights.reshape(-1), (0, (_N_PAD - n) * 8))

    mesh = plsc.VectorSubcoreMesh(core_axis_name="c", subcore_axis_name="s")
    f32 = jnp.float32
    i32 = jnp.int32
    cp = pltpu.CompilerParams()
    fields = pltpu.CompilerParams.__dataclass_fields__
    if "needs_layout_passes" in fields:
        cp = dataclasses.replace(cp, needs_layout_passes=False)
    if "use_tc_tiling_on_sc" in fields:
        cp = dataclasses.replace(cp, use_tc_tiling_on_sc=False)

    fn_rows = pl.kernel(
        _face_body,
        out_type=jax.ShapeDtypeStruct((_F_PAD, 16), f32),
        mesh=mesh,
        scratch_types=[
            pltpu.VMEM((_F_CH,), i32),
            pltpu.VMEM((_F_CH,), i32),
            pltpu.VMEM((_F_CH,), i32),
            pltpu.VMEM((_F_CH, 16), f32),
            pltpu.VMEM((_F_CH, 16), f32),
            pltpu.VMEM((_F_CH, 16), f32),
            pltpu.VMEM((_F_CH, 16), f32),
            pltpu.SemaphoreType.DMA,
        ],
        compiler_params=cp,
    )(p_rows, f1d[0], f1d[1], f1d[2])

    vn_rows = pl.kernel(
        _vert_body,
        out_type=jax.ShapeDtypeStruct((_N_PAD, 16), f32),
        mesh=mesh,
        scratch_types=[
            pltpu.VMEM((_V_CH * 8,), i32),
            pltpu.VMEM((_V_CH * 8,), i32),
            pltpu.VMEM((_V_CH * 8,), f32),
            pltpu.VMEM((_V_CH * 8,), f32),
            pltpu.VMEM((_V_CH * 8, 16), f32),
            pltpu.VMEM((_V_CH * 8, 16), f32),
            pltpu.VMEM((_V_CH, 16), f32),
            pltpu.VMEM((_V_CH, 16), f32),
            pltpu.SemaphoreType.DMA,
            pltpu.SemaphoreType.DMA,
            pltpu.SemaphoreType.DMA,
            pltpu.SemaphoreType.DMA,
        ],
        compiler_params=cp,
    )(fn_rows, vt1d, w_flat)

    pts2d = pl.pallas_call(
        lambda x_ref, o_ref: _center_body(x_ref, o_ref, n),
        out_shape=jax.ShapeDtypeStruct((n // 8, 128), f32),
    )(p_rows.reshape(n // 8, 128))

    pts = pts2d.reshape(n, 16)[:, : bs * 3].reshape(n, bs, 3).transpose(1, 0, 2)
    vn = vn_rows[:n, : bs * 3].reshape(n, bs, 3).transpose(1, 0, 2)
    rgb = jnp.ones((bs, n, 3), dt)
    return pts, vn, rgb


# speedup vs baseline: 2.5488x; 1.0009x over previous
import dataclasses

import jax
import jax.numpy as jnp
from jax import lax
from jax.experimental import pallas as pl
from jax.experimental.pallas import tpu as pltpu
from jax.experimental.pallas import tpu_sc as plsc

_L = 16
_NT = 32

_F_CH = 448
_F_CHUNKS = 14
_F_PER_TILE = _F_CH * _F_CHUNKS
_F_PAD = _F_PER_TILE * _NT

_V_CH = 224
_V_CHUNKS = 14
_V_PER_TILE = _V_CH * _V_CHUNKS
_N_PAD = _V_PER_TILE * _NT


def _c16(v):
    return jnp.full((_L,), v, jnp.int32)


def _rsqrt(s):
    i = plsc.bitcast(s, jnp.int32)
    y = plsc.bitcast(jnp.int32(0x5F3759DF) - (i >> 1), jnp.float32)
    xh = s * 0.5
    for _ in range(3):
        y = y * (1.5 - xh * y * y)
    return y


def _wid():
    return lax.axis_index("s") * 2 + lax.axis_index("c")


def _face_body(p_hbm, f0, f1, f2, fn_hbm,
               ia0, ib0, ic0, ia1, ib1, ic1,
               ra0, rb0, rc0, ra1, rb1, rc1, outb0, outb1,
               semi0, semi1, semg0, semg1):
    wid = _wid()
    iota = lax.iota(jnp.int32, _L)
    last = _F_CHUNKS - 1
    fs = (f0, f1, f2)

    def fire_idx(ch, idxs, sem):
        base = pl.multiple_of((wid * _F_CHUNKS + ch) * _F_CH, 8)
        for src, dst in zip(fs, idxs):
            pltpu.async_copy(src.at[pl.ds(base, _F_CH)], dst, sem)

    def wait_idx(idxs, sem):
        for src, dst in zip(fs, idxs):
            pltpu.make_async_copy(src.at[pl.ds(0, _F_CH)], dst, sem).wait()

    def fire_g(idxs, rs, sem):
        for idxr, rr in zip(idxs, rs):
            pltpu.async_copy(p_hbm.at[idxr], rr, sem)

    def wait_g(idxs, rs, sem):
        for idxr, rr in zip(idxs, rs):
            pltpu.make_async_copy(p_hbm.at[idxr], rr, sem).wait()

    def compute(ch, rs, outr):
        r0, r1, r2 = rs

        @pl.loop(0, _F_CH // _L)
        def _grp(g):
            riv = g * _L + iota
            a0 = [plsc.load_gather(r0, [riv, _c16(j)]) for j in range(12)]
            a1 = [plsc.load_gather(r1, [riv, _c16(j)]) for j in range(12)]
            a2 = [plsc.load_gather(r2, [riv, _c16(j)]) for j in range(12)]
            u = [a1[j] - a0[j] for j in range(12)]
            v = [a2[j] - a0[j] for j in range(12)]
            for b in range(4):
                X, Y, Z = 3 * b, 3 * b + 1, 3 * b + 2
                nx = u[Y] * v[Z] - u[Z] * v[Y]
                ny = u[Z] * v[X] - u[X] * v[Z]
                nz = u[X] * v[Y] - u[Y] * v[X]
                s = jnp.maximum(nx * nx + ny * ny + nz * nz, 1e-24)
                r = _rsqrt(s)
                plsc.store_scatter(outr, [riv, _c16(X)], nx * r)
                plsc.store_scatter(outr, [riv, _c16(Y)], ny * r)
                plsc.store_scatter(outr, [riv, _c16(Z)], nz * r)

        base = pl.multiple_of((wid * _F_CHUNKS + ch) * _F_CH, 8)
        pltpu.sync_copy(outr, fn_hbm.at[pl.ds(base, _F_CH)])

    i0 = (ia0, ib0, ic0)
    i1 = (ia1, ib1, ic1)
    rs0 = (ra0, rb0, rc0)
    rs1 = (ra1, rb1, rc1)

    fire_idx(0, i0, semi0)
    wait_idx(i0, semi0)
    fire_g(i0, rs0, semg0)
    fire_idx(1, i1, semi1)

    @pl.loop(0, _F_CHUNKS // 2)
    def _pair(i):
        e = i * 2
        wait_g(i0, rs0, semg0)
        wait_idx(i1, semi1)
        fire_g(i1, rs1, semg1)
        compute(e, rs0, outb0)
        fire_idx(jnp.minimum(e + 2, last), i0, semi0)
        wait_g(i1, rs1, semg1)
        wait_idx(i0, semi0)
        fire_g(i0, rs0, semg0)
        compute(e + 1, rs1, outb1)
        fire_idx(jnp.minimum(e + 3, last), i1, semi1)

    wait_g(i0, rs0, semg0)
    wait_idx(i1, semi1)


def _vert_body(fn_hbm, vt, w_hbm, vn_hbm,
               idx0, idx1, wb0, wb1, rows0, rows1, outb0, outb1,
               semi0, semi1, semg0, semg1):
    wid = _wid()
    iota = lax.iota(jnp.int32, _L)
    c8 = _V_CH * 8
    last = _V_CHUNKS - 1

    def fire_idxw(ch, idxr, wr, sem):
        cidx = wid * _V_CHUNKS + ch
        off = pl.multiple_of(cidx * c8, 8)
        pltpu.async_copy(vt.at[pl.ds(off, c8)], idxr, sem)
        pltpu.async_copy(w_hbm.at[pl.ds(off, c8)], wr, sem)

    def wait_idxw(idxr, wr, sem):
        pltpu.make_async_copy(vt.at[pl.ds(0, c8)], idxr, sem).wait()
        pltpu.make_async_copy(w_hbm.at[pl.ds(0, c8)], wr, sem).wait()

    def fire_g(idxr, rowsr, sem):
        pltpu.async_copy(fn_hbm.at[idxr], rowsr, sem)

    def wait_g(idxr, rowsr, sem):
        pltpu.make_async_copy(fn_hbm.at[idxr], rowsr, sem).wait()

    def compute(ch, wr, rowsr, outr):
        @pl.loop(0, _V_CH // _L)
        def _grp(g):
            riv = g * _L + iota
            rb = riv * 8
            rix = [rb + c for c in range(8)]
            ws = [plsc.load_gather(wr, [rix[c]]) for c in range(8)]
            for b in range(4):
                comp = []
                for k in range(3):
                    cj = _c16(3 * b + k)
                    t = ws[0] * plsc.load_gather(rowsr, [rix[0], cj])
                    for c in range(1, 8):
                        t = t + ws[c] * plsc.load_gather(rowsr, [rix[c], cj])
                    comp.append(t)
                s = jnp.maximum(
                    comp[0] * comp[0] + comp[1] * comp[1] + comp[2] * comp[2],
                    1e-24,
                )
                r = _rsqrt(s)
                for k in range(3):
                    plsc.store_scatter(outr, [riv, _c16(3 * b + k)],
                                       comp[k] * r)

        cidx = wid * _V_CHUNKS + ch
        off = pl.multiple_of(cidx * _V_CH, 8)
        pltpu.sync_copy(outr, vn_hbm.at[pl.ds(off, _V_CH)])

    fire_idxw(0, idx0, wb0, semi0)
    wait_idxw(idx0, wb0, semi0)
    fire_g(idx0, rows0, semg0)
    fire_idxw(1, idx1, wb1, semi1)

    @pl.loop(0, _V_CHUNKS // 2)
    def _pair(i):
        e = i * 2
        wait_g(idx0, rows0, semg0)
        wait_idxw(idx1, wb1, semi1)
        fire_g(idx1, rows1, semg1)
        compute(e, wb0, rows0, outb0)
        fire_idxw(jnp.minimum(e + 2, last), idx0, wb0, semi0)
        wait_g(idx1, rows1, semg1)
        wait_idxw(idx0, wb0, semi0)
        fire_g(idx0, rows0, semg0)
        compute(e + 1, wb1, rows1, outb1)
        fire_idxw(jnp.minimum(e + 3, last), idx1, wb1, semi1)

    wait_g(idx0, rows0, semg0)
    wait_idxw(idx1, wb1, semi1)


def _center_body(x_ref, o_ref, n):
    x = x_ref[...]
    s = jnp.sum(x, axis=0, keepdims=True)
    m = s[:, 0:16]
    for k in range(1, 8):
        m = m + s[:, 16 * k:16 * (k + 1)]
    m = m * (1.0 / n)
    o_ref[...] = x - jnp.concatenate([m] * 8, axis=1)


def kernel(points, faces, vert_tri_indices, vert_tri_weights):
    bs, n, _ = points.shape
    f = faces.shape[0]
    dt = points.dtype

    p_rows = jnp.transpose(points, (1, 0, 2)).reshape(n, bs * 3)
    p_rows = jnp.pad(p_rows, ((0, 0), (0, 16 - bs * 3)))
    f1d = [jnp.pad(faces[:, k], (0, _F_PAD - f)) for k in range(3)]
    vt1d = jnp.pad(vert_tri_indices.reshape(-1), (0, (_N_PAD - n) * 8))
    w_flat = jnp.pad(vert_tri_weights.reshape(-1), (0, (_N_PAD - n) * 8))

    mesh = plsc.VectorSubcoreMesh(core_axis_name="c", subcore_axis_name="s")
    f32 = jnp.float32
    i32 = jnp.int32
    cp = pltpu.CompilerParams()
    fields = pltpu.CompilerParams.__dataclass_fields__
    if "needs_layout_passes" in fields:
        cp = dataclasses.replace(cp, needs_layout_passes=False)
    if "use_tc_tiling_on_sc" in fields:
        cp = dataclasses.replace(cp, use_tc_tiling_on_sc=False)

    fn_rows = pl.kernel(
        _face_body,
        out_type=jax.ShapeDtypeStruct((_F_PAD, 16), f32),
        mesh=mesh,
        scratch_types=(
            [pltpu.VMEM((_F_CH,), i32)] * 6
            + [pltpu.VMEM((_F_CH, 16), f32)] * 8
            + [pltpu.SemaphoreType.DMA] * 4
        ),
        compiler_params=cp,
    )(p_rows, f1d[0], f1d[1], f1d[2])

    vn_rows = pl.kernel(
        _vert_body,
        out_type=jax.ShapeDtypeStruct((_N_PAD, 16), f32),
        mesh=mesh,
        scratch_types=[
            pltpu.VMEM((_V_CH * 8,), i32),
            pltpu.VMEM((_V_CH * 8,), i32),
            pltpu.VMEM((_V_CH * 8,), f32),
            pltpu.VMEM((_V_CH * 8,), f32),
            pltpu.VMEM((_V_CH * 8, 16), f32),
            pltpu.VMEM((_V_CH * 8, 16), f32),
            pltpu.VMEM((_V_CH, 16), f32),
            pltpu.VMEM((_V_CH, 16), f32),
            pltpu.SemaphoreType.DMA,
            pltpu.SemaphoreType.DMA,
            pltpu.SemaphoreType.DMA,
            pltpu.SemaphoreType.DMA,
        ],
        compiler_params=cp,
    )(fn_rows, vt1d, w_flat)

    pts2d = pl.pallas_call(
        lambda x_ref, o_ref: _center_body(x_ref, o_ref, n),
        out_shape=jax.ShapeDtypeStruct((n // 8, 128), f32),
    )(p_rows.reshape(n // 8, 128))

    pts = pts2d.reshape(n, 16)[:, : bs * 3].reshape(n, bs, 3).transpose(1, 0, 2)
    vn = vn_rows[:n, : bs * 3].reshape(n, bs, 3).transpose(1, 0, 2)
    rgb = jnp.ones((bs, n, 3), dt)
    return pts, vn, rgb
